# Initial kernel scaffold; baseline (speedup 1.0000x reference)
import jax
import jax.numpy as jnp
from jax.experimental import pallas as pl


def kernel(x, edge_index, Wl1, bl1, Wr1, Wl2, bl2, Wr2):
    raise NotImplementedError("write your pallas kernel here")


# speedup vs baseline: 8.4845x; 8.4845x over previous
import functools

import jax
import jax.numpy as jnp
from jax import lax
from jax.experimental import pallas as pl
from jax.experimental.pallas import tpu as pltpu
from jax.experimental.pallas import tpu_sc as plsc

NC = 2
NS = 16
NW = NC * NS


def _pick_chunk(edges_per_worker: int) -> int:
    for c in range(128, 0, -1):
        if edges_per_worker % c == 0 and (edges_per_worker // c) % 8 == 0:
            return c
    raise ValueError(f"no valid chunk size for {edges_per_worker}")


@functools.lru_cache(maxsize=None)
def _sc_aggregate(n: int, d: int, e: int, c: int, with_deg: bool):
    epw = e // NW
    assert epw * NW == e and epw % c == 0
    kpw = epw // c
    assert kpw % 8 == 0
    rpt = n // NS
    assert rpt * NS == n and rpt % 8 == 0
    eptd = e // NS
    kptd = eptd // c
    assert kptd * c == eptd and kptd % 8 == 0
    kg = next(g for g in range(min(kpw, 16), 0, -8)
              if kpw % g == 0 and kptd % g == 0)
    zc = next(z for z in range(min(c, 128) - min(c, 128) % 8, 0, -8)
              if rpt % z == 0)
    nz = rpt // zc
    nlane = d // 16

    mesh = plsc.VectorSubcoreMesh(core_axis_name="c", subcore_axis_name="s")
    out_type = [jax.ShapeDtypeStruct((NC, n, d), jnp.float32)]
    if with_deg:
        out_type.append(jax.ShapeDtypeStruct((n,), jnp.float32))

    scratch = [
        pltpu.VMEM((kg, c), jnp.int32),
        pltpu.VMEM((kg, c), jnp.int32),
        pltpu.VMEM((c, d), jnp.float32),
        pltpu.VMEM_SHARED((n, d), jnp.float32),
        pltpu.SemaphoreType.DMA,
    ]
    if with_deg:
        caux = -(-c // 16) * 16
        scratch += [
            pltpu.VMEM((caux,), jnp.float32),
            pltpu.VMEM_SHARED((n,), jnp.float32),
        ]

    def body(feats_hbm, src_hbm, dst_hbm, *refs):
        if with_deg:
            (outp_hbm, invd_hbm, src_v, dst_v, rows_v, agg_sh, sem,
             aux1d, deg_sh) = refs
        else:
            outp_hbm, src_v, dst_v, rows_v, agg_sh, sem = refs
        cid = lax.axis_index("c")
        sid = lax.axis_index("s")
        wid = cid * NS + sid

        zv = jnp.zeros((16,), jnp.float32)

        def zrow(r, _):
            for j in range(nlane):
                rows_v[r, pl.ds(j * 16, 16)] = zv
            return 0

        lax.fori_loop(0, c, zrow, 0)
        if with_deg:
            for j in range(caux // 16):
                aux1d[pl.ds(j * 16, 16)] = zv

        for t in range(nz):
            pltpu.sync_copy(rows_v.at[pl.ds(0, zc)],
                            agg_sh.at[pl.ds(sid * rpt + t * zc, zc)])
        if with_deg:
            @pl.when(cid == 0)
            def _():
                for t in range(-(-rpt // caux)):
                    w = min(caux, rpt - t * caux)
                    pltpu.sync_copy(
                        aux1d.at[pl.ds(0, w)],
                        deg_sh.at[pl.ds(sid * rpt + t * caux, w)])
            ov = jnp.ones((16,), jnp.float32)
            for j in range(caux // 16):
                aux1d[pl.ds(j * 16, 16)] = ov
        plsc.subcore_barrier()

        def chunk(k, _):
            pltpu.async_copy(feats_hbm.at[src_v.at[k]], rows_v, sem).wait()
            pltpu.sync_copy(rows_v, agg_sh.at[dst_v.at[k]], add=True)
            return 0

        for g in range(kpw // kg):
            pltpu.sync_copy(src_hbm.at[pl.ds(wid * kpw + g * kg, kg)], src_v)
            pltpu.sync_copy(dst_hbm.at[pl.ds(wid * kpw + g * kg, kg)], dst_v)
            lax.fori_loop(0, kg, chunk, 0)

        if with_deg:
            @pl.when(cid == 0)
            def _():
                def dchunk(k, _):
                    pltpu.sync_copy(aux1d.at[pl.ds(0, c)],
                                    deg_sh.at[dst_v.at[k]], add=True)
                    return 0

                for g in range(kptd // kg):
                    pltpu.sync_copy(
                        dst_hbm.at[pl.ds(sid * kptd + g * kg, kg)], dst_v)
                    lax.fori_loop(0, kg, dchunk, 0)

        plsc.subcore_barrier()

        pltpu.sync_copy(agg_sh.at[pl.ds(sid * rpt, rpt)],
                        outp_hbm.at[cid, pl.ds(sid * rpt, rpt)])

        if with_deg:
            @pl.when(cid == 0)
            def _():
                for t in range(-(-rpt // caux)):
                    w = min(caux, rpt - t * caux)
                    base = sid * rpt + t * caux
                    pltpu.sync_copy(deg_sh.at[pl.ds(base, w)],
                                    aux1d.at[pl.ds(0, w)])
                    for j in range(w // 16):
                        dv = aux1d[pl.ds(j * 16, 16)]
                        aux1d[pl.ds(j * 16, 16)] = (
                            1.0 / jnp.maximum(dv, 1.0))
                    pltpu.sync_copy(aux1d.at[pl.ds(0, w)],
                                    invd_hbm.at[pl.ds(base, w)])

    return pl.kernel(body, out_type=out_type, mesh=mesh,
                     scratch_types=scratch)


def _tc_dense(p, invd, xin, Wl, bl, Wr, relu: bool, block: int):
    n, d = xin.shape

    def body(p_ref, invd_ref, x_ref, wl_ref, bl_ref, wr_ref, o_ref):
        agg = (p_ref[0] + p_ref[1]) * invd_ref[...]
        y = (jnp.dot(agg, wl_ref[...], preferred_element_type=jnp.float32)
             + bl_ref[...]
             + jnp.dot(x_ref[...], wr_ref[...],
                       preferred_element_type=jnp.float32))
        if relu:
            y = jnp.maximum(y, 0.0)
        o_ref[...] = y

    grid = (n // block,)
    return pl.pallas_call(
        body,
        grid=grid,
        in_specs=[
            pl.BlockSpec((2, block, d), lambda i: (0, i, 0)),
            pl.BlockSpec((block, 1), lambda i: (i, 0)),
            pl.BlockSpec((block, d), lambda i: (i, 0)),
            pl.BlockSpec((d, d), lambda i: (0, 0)),
            pl.BlockSpec((1, d), lambda i: (0, 0)),
            pl.BlockSpec((d, d), lambda i: (0, 0)),
        ],
        out_specs=pl.BlockSpec((block, d), lambda i: (i, 0)),
        out_shape=jax.ShapeDtypeStruct((n, d), jnp.float32),
    )(p, invd, xin, Wl, bl, Wr)


def kernel(x, edge_index, Wl1, bl1, Wr1, Wl2, bl2, Wr2):
    n, d = x.shape
    e = edge_index.shape[1]
    ei = edge_index.astype(jnp.int32)
    c = _pick_chunk(e // NW)
    src2d = ei[0].reshape(e // c, c)
    dst2d = ei[1].reshape(e // c, c)

    align = NS * 128
    n2 = -(-n // align) * align
    x2 = jnp.pad(x, ((0, n2 - n), (0, 0))) if n2 != n else x
    block = next(b for b in range(1024, 0, -8) if n2 % b == 0)

    p1, invd1 = _sc_aggregate(n2, d, e, c, True)(x2, src2d, dst2d)
    invd = invd1.reshape(n2, 1)
    h = _tc_dense(p1, invd, x2, Wl1, bl1.reshape(1, d), Wr1,
                  relu=True, block=block)
    (p2,) = _sc_aggregate(n2, d, e, c, False)(h, src2d, dst2d)
    out = _tc_dense(p2, invd, h, Wl2, bl2.reshape(1, d), Wr2,
                    relu=False, block=block)
    return out[:n]


# speedup vs baseline: 10.9597x; 1.2917x over previous
import functools

import jax
import jax.numpy as jnp
from jax import lax
from jax.experimental import pallas as pl
from jax.experimental.pallas import tpu as pltpu
from jax.experimental.pallas import tpu_sc as plsc

NC = 2
NS = 16
NW = NC * NS


def _pick_chunk(edges_per_worker: int) -> int:
    for c in range(128, 0, -1):
        if edges_per_worker % c == 0 and (edges_per_worker // c) % 8 == 0:
            return c
    raise ValueError(f"no valid chunk size for {edges_per_worker}")


@functools.lru_cache(maxsize=None)
def _sc_aggregate(n: int, d: int, e: int, c: int, with_deg: bool):
    epw = e // NW
    assert epw * NW == e and epw % c == 0
    kpw = epw // c
    assert kpw % 8 == 0 and kpw % 2 == 0
    rpt = n // NS
    assert rpt * NS == n and rpt % 8 == 0
    kg = next(g for g in range(min(kpw, 40), 0, -8)
              if kpw % g == 0 and g % 2 == 0)
    zc = next(z for z in range(min(c, 128) - min(c, 128) % 8, 0, -8)
              if rpt % z == 0)
    nlane = d // 16
    caux = -(-c // 16) * 16
    row_bytes = c * d * 4
    deg_bytes = c * 4

    mesh = plsc.VectorSubcoreMesh(core_axis_name="c", subcore_axis_name="s")
    out_type = [jax.ShapeDtypeStruct((NC, n, d), jnp.float32)]
    if with_deg:
        out_type.append(jax.ShapeDtypeStruct((NC, n), jnp.float32))

    scratch = [
        pltpu.VMEM((kg, c), jnp.int32),
        pltpu.VMEM((kg, c), jnp.int32),
        pltpu.VMEM((c, d), jnp.float32),
        pltpu.VMEM((c, d), jnp.float32),
        pltpu.VMEM_SHARED((n, d), jnp.float32),
        pltpu.SemaphoreType.DMA,
        pltpu.SemaphoreType.DMA,
        pltpu.SemaphoreType.DMA,
        pltpu.SemaphoreType.DMA,
    ]
    if with_deg:
        scratch += [
            pltpu.VMEM((caux,), jnp.float32),
            pltpu.VMEM_SHARED((n,), jnp.float32),
            pltpu.SemaphoreType.DMA,
        ]

    def body(feats_hbm, src_hbm, dst_hbm, *refs):
        if with_deg:
            (outp_hbm, outdeg_hbm, src_v, dst_v, rows0, rows1, agg_sh,
             gsem0, gsem1, ssem0, ssem1, aux1d, deg_sh, dsem) = refs
        else:
            (outp_hbm, src_v, dst_v, rows0, rows1, agg_sh,
             gsem0, gsem1, ssem0, ssem1) = refs
        rows = (rows0, rows1)
        gsem = (gsem0, gsem1)
        ssem = (ssem0, ssem1)
        cid = lax.axis_index("c")
        sid = lax.axis_index("s")
        wid = cid * NS + sid

        zv = jnp.zeros((16,), jnp.float32)

        def zrow(r, _):
            for j in range(nlane):
                rows0[r, pl.ds(j * 16, 16)] = zv
            return 0

        lax.fori_loop(0, c, zrow, 0)
        if with_deg:
            ov = jnp.ones((16,), jnp.float32)
            for j in range(caux // 16):
                aux1d[pl.ds(j * 16, 16)] = zv

        for t in range(rpt // zc):
            pltpu.sync_copy(rows0.at[pl.ds(0, zc)],
                            agg_sh.at[pl.ds(sid * rpt + t * zc, zc)])
        if with_deg:
            for t in range(-(-rpt // caux)):
                w = min(caux, rpt - t * caux)
                pltpu.sync_copy(
                    aux1d.at[pl.ds(0, w)],
                    deg_sh.at[pl.ds(sid * rpt + t * caux, w)])
            for j in range(caux // 16):
                aux1d[pl.ds(j * 16, 16)] = ov
        plsc.subcore_barrier()

        def gather(k, b):
            return pltpu.async_copy(feats_hbm.at[src_v.at[k]], rows[b],
                                    gsem[b])

        def scatter(k, b):
            return pltpu.async_copy(rows[b], agg_sh.at[dst_v.at[k]],
                                    ssem[b], add=True)

        def wait_g(b):
            pltpu.make_async_copy(feats_hbm.at[src_v.at[0]], rows[b],
                                  gsem[b]).wait()

        def wait_s(b):
            pltpu.make_async_copy(rows[b], agg_sh.at[dst_v.at[0]],
                                  ssem[b]).wait()

        npairs = kg // 2
        for g in range(kpw // kg):
            pltpu.sync_copy(src_hbm.at[pl.ds(wid * kpw + g * kg, kg)], src_v)
            pltpu.sync_copy(dst_hbm.at[pl.ds(wid * kpw + g * kg, kg)], dst_v)
            gather(0, 0)
            gather(1, 1)

            def pair(t, _):
                k0 = 2 * t
                wait_g(0)
                scatter(k0, 0)
                if with_deg:
                    pltpu.async_copy(aux1d.at[pl.ds(0, c)],
                                     deg_sh.at[dst_v.at[k0]], dsem,
                                     add=True)
                wait_g(1)
                scatter(k0 + 1, 1)
                if with_deg:
                    pltpu.async_copy(aux1d.at[pl.ds(0, c)],
                                     deg_sh.at[dst_v.at[k0 + 1]], dsem,
                                     add=True)

                @pl.when(t < npairs - 1)
                def _():
                    wait_s(0)
                    gather(k0 + 2, 0)
                    wait_s(1)
                    gather(k0 + 3, 1)

                return 0

            lax.fori_loop(0, npairs, pair, 0)
            wait_s(0)
            wait_s(1)
            if with_deg:
                def ddrain(t, _):
                    pltpu.make_async_copy(aux1d.at[pl.ds(0, c)],
                                          deg_sh.at[dst_v.at[0]],
                                          dsem).wait()
                    return 0

                lax.fori_loop(0, kg, ddrain, 0)

        plsc.subcore_barrier()

        pltpu.sync_copy(agg_sh.at[pl.ds(sid * rpt, rpt)],
                        outp_hbm.at[cid, pl.ds(sid * rpt, rpt)])
        if with_deg:
            pltpu.sync_copy(deg_sh.at[pl.ds(sid * rpt, rpt)],
                            outdeg_hbm.at[cid, pl.ds(sid * rpt, rpt)])

    return pl.kernel(body, out_type=out_type, mesh=mesh,
                     scratch_types=scratch)


def _tc_layer1(p, degp, xin, Wl, bl, Wr, block: int):
    n, d = xin.shape

    def body(p_ref, deg_ref, x_ref, wl_ref, bl_ref, wr_ref, o_ref, inv_ref):
        dsum = deg_ref[0] + deg_ref[1]
        invd = 1.0 / jnp.maximum(dsum, 1.0)
        agg = (p_ref[0] + p_ref[1]) * invd
        y = (jnp.dot(agg, wl_ref[...], preferred_element_type=jnp.float32)
             + bl_ref[...]
             + jnp.dot(x_ref[...], wr_ref[...],
                       preferred_element_type=jnp.float32))
        o_ref[...] = jnp.maximum(y, 0.0)
        inv_ref[...] = invd

    return pl.pallas_call(
        body,
        grid=(n // block,),
        in_specs=[
            pl.BlockSpec((2, block, d), lambda i: (0, i, 0)),
            pl.BlockSpec((2, block, 1), lambda i: (0, i, 0)),
            pl.BlockSpec((block, d), lambda i: (i, 0)),
            pl.BlockSpec((d, d), lambda i: (0, 0)),
            pl.BlockSpec((1, d), lambda i: (0, 0)),
            pl.BlockSpec((d, d), lambda i: (0, 0)),
        ],
        out_specs=[
            pl.BlockSpec((block, d), lambda i: (i, 0)),
            pl.BlockSpec((block, 1), lambda i: (i, 0)),
        ],
        out_shape=[
            jax.ShapeDtypeStruct((n, d), jnp.float32),
            jax.ShapeDtypeStruct((n, 1), jnp.float32),
        ],
    )(p, degp, xin, Wl, bl, Wr)


def _tc_layer2(p, invd, xin, Wl, bl, Wr, block: int):
    n, d = xin.shape

    def body(p_ref, inv_ref, x_ref, wl_ref, bl_ref, wr_ref, o_ref):
        agg = (p_ref[0] + p_ref[1]) * inv_ref[...]
        o_ref[...] = (
            jnp.dot(agg, wl_ref[...], preferred_element_type=jnp.float32)
            + bl_ref[...]
            + jnp.dot(x_ref[...], wr_ref[...],
                      preferred_element_type=jnp.float32))

    return pl.pallas_call(
        body,
        grid=(n // block,),
        in_specs=[
            pl.BlockSpec((2, block, d), lambda i: (0, i, 0)),
            pl.BlockSpec((block, 1), lambda i: (i, 0)),
            pl.BlockSpec((block, d), lambda i: (i, 0)),
            pl.BlockSpec((d, d), lambda i: (0, 0)),
            pl.BlockSpec((1, d), lambda i: (0, 0)),
            pl.BlockSpec((d, d), lambda i: (0, 0)),
        ],
        out_specs=pl.BlockSpec((block, d), lambda i: (i, 0)),
        out_shape=jax.ShapeDtypeStruct((n, d), jnp.float32),
    )(p, invd, xin, Wl, bl, Wr)


def kernel(x, edge_index, Wl1, bl1, Wr1, Wl2, bl2, Wr2):
    n, d = x.shape
    e = edge_index.shape[1]
    ei = edge_index.astype(jnp.int32)
    c = _pick_chunk(e // NW)
    src2d = ei[0].reshape(e // c, c)
    dst2d = ei[1].reshape(e // c, c)

    align = NS * 128
    n2 = -(-n // align) * align
    x2 = jnp.pad(x, ((0, n2 - n), (0, 0))) if n2 != n else x
    block = next(b for b in range(1024, 0, -8) if n2 % b == 0)

    p1, degp = _sc_aggregate(n2, d, e, c, True)(x2, src2d, dst2d)
    h, invd = _tc_layer1(p1, degp.reshape(NC, n2, 1), x2, Wl1,
                         bl1.reshape(1, d), Wr1, block=block)
    (p2,) = _sc_aggregate(n2, d, e, c, False)(h, src2d, dst2d)
    out = _tc_layer2(p2, invd, h, Wl2, bl2.reshape(1, d), Wr2, block=block)
    return out[:n]


# speedup vs baseline: 11.1892x; 1.0209x over previous
import functools

import jax
import jax.numpy as jnp
from jax import lax
from jax.experimental import pallas as pl
from jax.experimental.pallas import tpu as pltpu
from jax.experimental.pallas import tpu_sc as plsc

NC = 2
NS = 16
NW = NC * NS


def _pick_chunk(edges_per_worker: int) -> int:
    for c in range(128, 0, -1):
        if edges_per_worker % c == 0 and (edges_per_worker // c) % 8 == 0:
            return c
    raise ValueError(f"no valid chunk size for {edges_per_worker}")


@functools.lru_cache(maxsize=None)
def _sc_aggregate(n: int, nf: int, d: int, e: int, c: int, with_deg: bool):
    epw = e // NW
    assert epw * NW == e and epw % c == 0
    kpw = epw // c
    assert kpw % 8 == 0 and kpw % 2 == 0
    rpt = n // NS
    assert rpt * NS == n and rpt % 8 == 0
    kg = next(g for g in range(min(kpw, 40), 0, -8)
              if kpw % g == 0 and g % 2 == 0)
    zc = next(z for z in range(min(c, 128) - min(c, 128) % 8, 0, -8)
              if rpt % z == 0)
    nlane = d // 16
    caux = -(-c // 16) * 16
    row_bytes = c * d * 4
    deg_bytes = c * 4

    mesh = plsc.VectorSubcoreMesh(core_axis_name="c", subcore_axis_name="s")
    out_type = [jax.ShapeDtypeStruct((NC, n, d), jnp.float32)]
    if with_deg:
        out_type.append(jax.ShapeDtypeStruct((NC, n), jnp.float32))

    scratch = [
        pltpu.VMEM((kg, c), jnp.int32),
        pltpu.VMEM((kg, c), jnp.int32),
        pltpu.VMEM((c, d), jnp.float32),
        pltpu.VMEM((c, d), jnp.float32),
        pltpu.VMEM_SHARED((n, d), jnp.float32),
        pltpu.SemaphoreType.DMA,
        pltpu.SemaphoreType.DMA,
        pltpu.SemaphoreType.DMA,
        pltpu.SemaphoreType.DMA,
    ]
    if with_deg:
        scratch += [
            pltpu.VMEM((caux,), jnp.float32),
            pltpu.VMEM_SHARED((n,), jnp.float32),
            pltpu.SemaphoreType.DMA,
        ]

    def body(feats_hbm, src_hbm, dst_hbm, *refs):
        if with_deg:
            (outp_hbm, outdeg_hbm, src_v, dst_v, rows0, rows1, agg_sh,
             gsem0, gsem1, ssem0, ssem1, aux1d, deg_sh, dsem) = refs
        else:
            (outp_hbm, src_v, dst_v, rows0, rows1, agg_sh,
             gsem0, gsem1, ssem0, ssem1) = refs
        rows = (rows0, rows1)
        gsem = (gsem0, gsem1)
        ssem = (ssem0, ssem1)
        cid = lax.axis_index("c")
        sid = lax.axis_index("s")
        wid = cid * NS + sid

        zv = jnp.zeros((16,), jnp.float32)

        def zrow(r, _):
            for j in range(nlane):
                rows0[r, pl.ds(j * 16, 16)] = zv
            return 0

        lax.fori_loop(0, c, zrow, 0)
        if with_deg:
            ov = jnp.ones((16,), jnp.float32)
            for j in range(caux // 16):
                aux1d[pl.ds(j * 16, 16)] = zv

        for t in range(rpt // zc):
            pltpu.sync_copy(rows0.at[pl.ds(0, zc)],
                            agg_sh.at[pl.ds(sid * rpt + t * zc, zc)])
        if with_deg:
            for t in range(-(-rpt // caux)):
                w = min(caux, rpt - t * caux)
                pltpu.sync_copy(
                    aux1d.at[pl.ds(0, w)],
                    deg_sh.at[pl.ds(sid * rpt + t * caux, w)])
            for j in range(caux // 16):
                aux1d[pl.ds(j * 16, 16)] = ov
        plsc.subcore_barrier()

        def gather(k, b):
            return pltpu.async_copy(feats_hbm.at[src_v.at[k]], rows[b],
                                    gsem[b])

        def scatter(k, b):
            return pltpu.async_copy(rows[b], agg_sh.at[dst_v.at[k]],
                                    ssem[b], add=True)

        def wait_g(b):
            pltpu.make_async_copy(feats_hbm.at[src_v.at[0]], rows[b],
                                  gsem[b]).wait()

        def wait_s(b):
            pltpu.make_async_copy(rows[b], agg_sh.at[dst_v.at[0]],
                                  ssem[b]).wait()

        npairs = kg // 2
        for g in range(kpw // kg):
            pltpu.sync_copy(src_hbm.at[pl.ds(wid * kpw + g * kg, kg)], src_v)
            pltpu.sync_copy(dst_hbm.at[pl.ds(wid * kpw + g * kg, kg)], dst_v)
            gather(0, 0)
            gather(1, 1)

            def pair(t, _):
                k0 = 2 * t
                wait_g(0)
                scatter(k0, 0)
                if with_deg:
                    pltpu.async_copy(aux1d.at[pl.ds(0, c)],
                                     deg_sh.at[dst_v.at[k0]], dsem,
                                     add=True)
                wait_g(1)
                scatter(k0 + 1, 1)
                if with_deg:
                    pltpu.async_copy(aux1d.at[pl.ds(0, c)],
                                     deg_sh.at[dst_v.at[k0 + 1]], dsem,
                                     add=True)

                @pl.when(t < npairs - 1)
                def _():
                    wait_s(0)
                    gather(k0 + 2, 0)
                    wait_s(1)
                    gather(k0 + 3, 1)

                return 0

            lax.fori_loop(0, npairs, pair, 0)
            wait_s(0)
            wait_s(1)
            if with_deg:
                def ddrain(t, _):
                    pltpu.make_async_copy(aux1d.at[pl.ds(0, c)],
                                          deg_sh.at[dst_v.at[0]],
                                          dsem).wait()
                    return 0

                lax.fori_loop(0, kg, ddrain, 0)

        plsc.subcore_barrier()

        pltpu.sync_copy(agg_sh.at[pl.ds(sid * rpt, rpt)],
                        outp_hbm.at[cid, pl.ds(sid * rpt, rpt)])
        if with_deg:
            pltpu.sync_copy(deg_sh.at[pl.ds(sid * rpt, rpt)],
                            outdeg_hbm.at[cid, pl.ds(sid * rpt, rpt)])

    return pl.kernel(body, out_type=out_type, mesh=mesh,
                     scratch_types=scratch)


def _tc_layer1(p, degp, xin, Wl, bl, Wr, block: int):
    n, d = xin.shape

    def body(p_ref, deg_ref, x_ref, wl_ref, bl_ref, wr_ref, o_ref, inv_ref):
        dsum = deg_ref[0] + deg_ref[1]
        invd = 1.0 / jnp.maximum(dsum, 1.0)
        agg = (p_ref[0] + p_ref[1]) * invd
        y = (jnp.dot(agg, wl_ref[...], preferred_element_type=jnp.float32)
             + bl_ref[...]
             + jnp.dot(x_ref[...], wr_ref[...],
                       preferred_element_type=jnp.float32))
        o_ref[...] = jnp.maximum(y, 0.0)
        inv_ref[...] = invd

    return pl.pallas_call(
        body,
        grid=(n // block,),
        in_specs=[
            pl.BlockSpec((2, block, d), lambda i: (0, i, 0)),
            pl.BlockSpec((2, block, 1), lambda i: (0, i, 0)),
            pl.BlockSpec((block, d), lambda i: (i, 0)),
            pl.BlockSpec((d, d), lambda i: (0, 0)),
            pl.BlockSpec((1, d), lambda i: (0, 0)),
            pl.BlockSpec((d, d), lambda i: (0, 0)),
        ],
        out_specs=[
            pl.BlockSpec((block, d), lambda i: (i, 0)),
            pl.BlockSpec((block, 1), lambda i: (i, 0)),
        ],
        out_shape=[
            jax.ShapeDtypeStruct((n, d), jnp.float32),
            jax.ShapeDtypeStruct((n, 1), jnp.float32),
        ],
    )(p, degp, xin, Wl, bl, Wr)


def _tc_layer2(p, invd, xin, Wl, bl, Wr, block: int):
    n, d = xin.shape

    def body(p_ref, inv_ref, x_ref, wl_ref, bl_ref, wr_ref, o_ref):
        agg = (p_ref[0] + p_ref[1]) * inv_ref[...]
        o_ref[...] = (
            jnp.dot(agg, wl_ref[...], preferred_element_type=jnp.float32)
            + bl_ref[...]
            + jnp.dot(x_ref[...], wr_ref[...],
                      preferred_element_type=jnp.float32))

    return pl.pallas_call(
        body,
        grid=(n // block,),
        in_specs=[
            pl.BlockSpec((2, block, d), lambda i: (0, i, 0)),
            pl.BlockSpec((block, 1), lambda i: (i, 0)),
            pl.BlockSpec((block, d), lambda i: (i, 0)),
            pl.BlockSpec((d, d), lambda i: (0, 0)),
            pl.BlockSpec((1, d), lambda i: (0, 0)),
            pl.BlockSpec((d, d), lambda i: (0, 0)),
        ],
        out_specs=pl.BlockSpec((block, d), lambda i: (i, 0)),
        out_shape=jax.ShapeDtypeStruct((n, d), jnp.float32),
    )(p, invd, xin, Wl, bl, Wr)


def kernel(x, edge_index, Wl1, bl1, Wr1, Wl2, bl2, Wr2):
    n, d = x.shape
    e = edge_index.shape[1]
    ei = edge_index.astype(jnp.int32)
    c = _pick_chunk(e // NW)
    src2d = ei[0].reshape(e // c, c)
    dst2d = ei[1].reshape(e // c, c)

    align = NS * 128
    n2 = -(-n // align) * align
    block = next(b for b in range(1024, 0, -8) if n % b == 0)

    p1, degp = _sc_aggregate(n2, n, d, e, c, True)(x, src2d, dst2d)
    h, invd = _tc_layer1(p1, degp.reshape(NC, n2, 1), x, Wl1,
                         bl1.reshape(1, d), Wr1, block=block)
    (p2,) = _sc_aggregate(n2, n, d, e, c, False)(h, src2d, dst2d)
    return _tc_layer2(p2, invd, h, Wl2, bl2.reshape(1, d), Wr2, block=block)
